# baseline (speedup 1.0000x reference)
import jax
import jax.numpy as jnp
from jax import lax
from jax.experimental import pallas as pl
from jax.experimental.pallas import tpu as pltpu
from jax.experimental.pallas import tpu_sc as plsc

B = 16
S = 4096
D = 1024
LANES = 16
RB = 512


def _tc_count_body(x_ref, cnt_ref):
    r = pl.program_id(1)
    x = x_ref[0]
    rows = lax.broadcasted_iota(jnp.int32, (RB, 128), 0)
    nz = jnp.where((x != 0.0) & (rows % 2 == 1), 1, 0)
    part = jnp.broadcast_to(jnp.sum(nz, axis=0)[None, :], (8, 128))

    @pl.when(r == 0)
    def _():
        cnt_ref[0] = part

    @pl.when(r != 0)
    def _():
        cnt_ref[0] += part


def _sc_gather_body(sent_hbm, cnt_hbm, a_hat_hbm, a_n_hbm,
                    cnt_v, block_v, rows_sh):
    c = lax.axis_index("c")
    s = lax.axis_index("s")
    b = c * 8 + s // 2
    h = s % 2

    pltpu.sync_copy(cnt_hbm.at[b], cnt_v)
    lane = lax.iota(jnp.int32, LANES)
    count = jnp.sum(jnp.where(lane == 0, cnt_v[0, 0:LANES], 0))

    r = count - 2 + h
    r = jnp.where(r < 0, r + S, r)
    rb = pl.multiple_of((r // 8) * 8, 8)
    pltpu.sync_copy(sent_hbm.at[b, pl.ds(rb, 8), :], block_v)
    pltpu.sync_copy(block_v.at[r - rb], rows_sh.at[h, s // 2])
    plsc.subcore_barrier()

    out0 = pl.multiple_of(c * 8, 8)

    @pl.when(s == 0)
    def _():
        pltpu.sync_copy(rows_sh.at[0], a_hat_hbm.at[pl.ds(out0, 8), :])

    @pl.when(s == 1)
    def _():
        pltpu.sync_copy(rows_sh.at[1], a_n_hbm.at[pl.ds(out0, 8), :])


@jax.jit
def kernel(sent_emb, entity_emb):
    del entity_emb

    counts = pl.pallas_call(
        _tc_count_body,
        grid=(B, S // RB),
        in_specs=[pl.BlockSpec((1, RB, 128), lambda b, r: (b, r, 0))],
        out_specs=pl.BlockSpec((1, 8, 128), lambda b, r: (b, 0, 0)),
        out_shape=jax.ShapeDtypeStruct((B, 8, 128), jnp.int32),
    )(sent_emb)

    out_row = jax.ShapeDtypeStruct((B, D), jnp.float32)
    sc_call = pl.kernel(
        _sc_gather_body,
        out_type=(out_row, out_row),
        mesh=plsc.VectorSubcoreMesh(core_axis_name="c", subcore_axis_name="s"),
        scratch_types=[
            pltpu.VMEM((8, 128), jnp.int32),
            pltpu.VMEM((8, D), jnp.float32),
            pltpu.VMEM_SHARED((2, 8, D), jnp.float32),
        ],
        compiler_params=pltpu.CompilerParams(needs_layout_passes=False),
    )
    sent_a_hat_n, sent_a_n = sc_call(sent_emb, counts)
    return sent_a_hat_n, sent_a_n, jnp.asarray(0.0, dtype=jnp.float32)


# speedup vs baseline: 1.8924x; 1.8924x over previous
import jax
import jax.numpy as jnp
from jax import lax
from jax.experimental import pallas as pl
from jax.experimental.pallas import tpu as pltpu
from jax.experimental.pallas import tpu_sc as plsc

B = 16
S = 4096
D = 1024
LANES = 16
RB = 512


def _tc_count_body(x0_ref, x1_ref, x2_ref, x3_ref, cnt_ref):
    r = pl.program_id(1)
    rows = lax.broadcasted_iota(jnp.int32, (RB, 128), 0)
    nz = jnp.where((x0_ref[0] != 0.0) & (rows % 2 == 1), 1, 0)
    nz += jnp.where((x1_ref[0] != 0.0) & (rows % 2 == 1), 1, 0)
    nz += jnp.where((x2_ref[0] != 0.0) & (rows % 2 == 1), 1, 0)
    nz += jnp.where((x3_ref[0] != 0.0) & (rows % 2 == 1), 1, 0)
    part = jnp.broadcast_to(jnp.sum(nz, axis=0)[None, :], (8, 128))

    @pl.when(r == 0)
    def _():
        cnt_ref[0] = part

    @pl.when(r != 0)
    def _():
        cnt_ref[0] += part


def _sc_gather_body(sent_hbm, cnt_hbm, a_hat_hbm, a_n_hbm,
                    cnt_v, block_v, rows_sh):
    c = lax.axis_index("c")
    s = lax.axis_index("s")
    b = c * 8 + s // 2
    h = s % 2

    pltpu.sync_copy(cnt_hbm.at[b], cnt_v)
    lane = lax.iota(jnp.int32, LANES)
    count = jnp.sum(jnp.where(lane == 0, cnt_v[0, 0:LANES], 0))

    r = count - 2 + h
    r = jnp.where(r < 0, r + S, r)
    rb = pl.multiple_of((r // 8) * 8, 8)
    pltpu.sync_copy(sent_hbm.at[b, pl.ds(rb, 8), :], block_v)
    pltpu.sync_copy(block_v.at[r - rb], rows_sh.at[h, s // 2])
    plsc.subcore_barrier()

    out0 = pl.multiple_of(c * 8, 8)

    @pl.when(s == 0)
    def _():
        pltpu.sync_copy(rows_sh.at[0], a_hat_hbm.at[pl.ds(out0, 8), :])

    @pl.when(s == 1)
    def _():
        pltpu.sync_copy(rows_sh.at[1], a_n_hbm.at[pl.ds(out0, 8), :])


@jax.jit
def kernel(sent_emb, entity_emb):
    del entity_emb

    counts = pl.pallas_call(
        _tc_count_body,
        grid=(B, S // RB // 4),
        in_specs=[
            pl.BlockSpec((1, RB, 128), lambda b, r, j=j: (b, 4 * r + j, 0))
            for j in range(4)
        ],
        out_specs=pl.BlockSpec((1, 8, 128), lambda b, r: (b, 0, 0)),
        out_shape=jax.ShapeDtypeStruct((B, 8, 128), jnp.int32),
    )(sent_emb, sent_emb, sent_emb, sent_emb)

    out_row = jax.ShapeDtypeStruct((B, D), jnp.float32)
    sc_call = pl.kernel(
        _sc_gather_body,
        out_type=(out_row, out_row),
        mesh=plsc.VectorSubcoreMesh(core_axis_name="c", subcore_axis_name="s"),
        scratch_types=[
            pltpu.VMEM((8, 128), jnp.int32),
            pltpu.VMEM((8, D), jnp.float32),
            pltpu.VMEM_SHARED((2, 8, D), jnp.float32),
        ],
        compiler_params=pltpu.CompilerParams(needs_layout_passes=False),
    )
    sent_a_hat_n, sent_a_n = sc_call(sent_emb, counts)
    return sent_a_hat_n, sent_a_n, jnp.asarray(0.0, dtype=jnp.float32)


# speedup vs baseline: 2.2860x; 1.2080x over previous
import jax
import jax.numpy as jnp
from jax import lax
from jax.experimental import pallas as pl
from jax.experimental.pallas import tpu as pltpu
from jax.experimental.pallas import tpu_sc as plsc

B = 16
S = 4096
D = 1024
LANES = 16
RB = 512


N_STREAMS = 8


def _tc_count_body(*refs):
    (*x_refs, cnt_ref) = refs
    r = pl.program_id(1)
    rows = lax.broadcasted_iota(jnp.int32, (RB, 128), 0)
    odd = rows % 2 == 1
    nz = jnp.zeros((RB, 128), jnp.int32)
    for x_ref in x_refs:
        nz += jnp.where((x_ref[0] != 0.0) & odd, 1, 0)
    part = jnp.broadcast_to(jnp.sum(nz, axis=0)[None, :], (8, 128))

    @pl.when(r == 0)
    def _():
        cnt_ref[0] = part

    @pl.when(r != 0)
    def _():
        cnt_ref[0] += part


def _sc_gather_body(sent_hbm, cnt_hbm, a_hat_hbm, a_n_hbm,
                    cnt_v, block_v, rows_sh):
    c = lax.axis_index("c")
    s = lax.axis_index("s")
    b = c * 8 + s // 2
    h = s % 2

    pltpu.sync_copy(cnt_hbm.at[b], cnt_v)
    lane = lax.iota(jnp.int32, LANES)
    count = jnp.sum(jnp.where(lane == 0, cnt_v[0, 0:LANES], 0))

    r = count - 2 + h
    r = jnp.where(r < 0, r + S, r)
    rb = pl.multiple_of((r // 8) * 8, 8)
    pltpu.sync_copy(sent_hbm.at[b, pl.ds(rb, 8), :], block_v)
    pltpu.sync_copy(block_v.at[r - rb], rows_sh.at[h, s // 2])
    plsc.subcore_barrier()

    out0 = pl.multiple_of(c * 8, 8)

    @pl.when(s == 0)
    def _():
        pltpu.sync_copy(rows_sh.at[0], a_hat_hbm.at[pl.ds(out0, 8), :])

    @pl.when(s == 1)
    def _():
        pltpu.sync_copy(rows_sh.at[1], a_n_hbm.at[pl.ds(out0, 8), :])


@jax.jit
def kernel(sent_emb, entity_emb):
    del entity_emb

    counts = pl.pallas_call(
        _tc_count_body,
        grid=(B, S // RB // N_STREAMS),
        in_specs=[
            pl.BlockSpec((1, RB, 128),
                         lambda b, r, j=j: (b, N_STREAMS * r + j, 0))
            for j in range(N_STREAMS)
        ],
        out_specs=pl.BlockSpec((1, 8, 128), lambda b, r: (b, 0, 0)),
        out_shape=jax.ShapeDtypeStruct((B, 8, 128), jnp.int32),
    )(*([sent_emb] * N_STREAMS))

    out_row = jax.ShapeDtypeStruct((B, D), jnp.float32)
    sc_call = pl.kernel(
        _sc_gather_body,
        out_type=(out_row, out_row),
        mesh=plsc.VectorSubcoreMesh(core_axis_name="c", subcore_axis_name="s"),
        scratch_types=[
            pltpu.VMEM((8, 128), jnp.int32),
            pltpu.VMEM((8, D), jnp.float32),
            pltpu.VMEM_SHARED((2, 8, D), jnp.float32),
        ],
        compiler_params=pltpu.CompilerParams(needs_layout_passes=False),
    )
    sent_a_hat_n, sent_a_n = sc_call(sent_emb, counts)
    return sent_a_hat_n, sent_a_n, jnp.asarray(0.0, dtype=jnp.float32)


# speedup vs baseline: 2.2890x; 1.0013x over previous
import jax
import jax.numpy as jnp
from jax import lax
from jax.experimental import pallas as pl
from jax.experimental.pallas import tpu as pltpu
from jax.experimental.pallas import tpu_sc as plsc

B = 16
S = 4096
D = 1024
LANES = 16
RB = 256


N_STREAMS = 16


def _tc_count_body(*refs):
    (*x_refs, cnt_ref) = refs
    r = pl.program_id(1)
    rows = lax.broadcasted_iota(jnp.int32, (RB, 128), 0)
    odd = rows % 2 == 1
    nz = jnp.zeros((RB, 128), jnp.int32)
    for x_ref in x_refs:
        nz += jnp.where((x_ref[0] != 0.0) & odd, 1, 0)
    part = jnp.broadcast_to(jnp.sum(nz, axis=0)[None, :], (8, 128))

    @pl.when(r == 0)
    def _():
        cnt_ref[0] = part

    @pl.when(r != 0)
    def _():
        cnt_ref[0] += part


def _sc_gather_body(sent_hbm, cnt_hbm, a_hat_hbm, a_n_hbm,
                    cnt_v, block_v, rows_sh):
    c = lax.axis_index("c")
    s = lax.axis_index("s")
    b = c * 8 + s // 2
    h = s % 2

    pltpu.sync_copy(cnt_hbm.at[b], cnt_v)
    lane = lax.iota(jnp.int32, LANES)
    count = jnp.sum(jnp.where(lane == 0, cnt_v[0, 0:LANES], 0))

    r = count - 2 + h
    r = jnp.where(r < 0, r + S, r)
    rb = pl.multiple_of((r // 8) * 8, 8)
    pltpu.sync_copy(sent_hbm.at[b, pl.ds(rb, 8), :], block_v)
    pltpu.sync_copy(block_v.at[r - rb], rows_sh.at[h, s // 2])
    plsc.subcore_barrier()

    out0 = pl.multiple_of(c * 8, 8)

    @pl.when(s == 0)
    def _():
        pltpu.sync_copy(rows_sh.at[0], a_hat_hbm.at[pl.ds(out0, 8), :])

    @pl.when(s == 1)
    def _():
        pltpu.sync_copy(rows_sh.at[1], a_n_hbm.at[pl.ds(out0, 8), :])


@jax.jit
def kernel(sent_emb, entity_emb):
    del entity_emb

    counts = pl.pallas_call(
        _tc_count_body,
        grid=(B, S // RB // N_STREAMS),
        in_specs=[
            pl.BlockSpec((1, RB, 128),
                         lambda b, r, j=j: (b, N_STREAMS * r + j, 0))
            for j in range(N_STREAMS)
        ],
        out_specs=pl.BlockSpec((1, 8, 128), lambda b, r: (b, 0, 0)),
        out_shape=jax.ShapeDtypeStruct((B, 8, 128), jnp.int32),
    )(*([sent_emb] * N_STREAMS))

    out_row = jax.ShapeDtypeStruct((B, D), jnp.float32)
    sc_call = pl.kernel(
        _sc_gather_body,
        out_type=(out_row, out_row),
        mesh=plsc.VectorSubcoreMesh(core_axis_name="c", subcore_axis_name="s"),
        scratch_types=[
            pltpu.VMEM((8, 128), jnp.int32),
            pltpu.VMEM((8, D), jnp.float32),
            pltpu.VMEM_SHARED((2, 8, D), jnp.float32),
        ],
        compiler_params=pltpu.CompilerParams(needs_layout_passes=False),
    )
    sent_a_hat_n, sent_a_n = sc_call(sent_emb, counts)
    return sent_a_hat_n, sent_a_n, jnp.asarray(0.0, dtype=jnp.float32)


# speedup vs baseline: 2.3809x; 1.0401x over previous
import jax
import jax.numpy as jnp
from jax import lax
from jax.experimental import pallas as pl
from jax.experimental.pallas import tpu as pltpu
from jax.experimental.pallas import tpu_sc as plsc

B = 16
S = 4096
D = 1024
LANES = 16
RSC = 2048
CH = 512
N_CH = (RSC // 2) // CH
STEPS = (CH // 2) // LANES
RB = 256
N_STREAMS = 8
TC_BLK0 = RSC // RB


def _sc_count_body(sent_hbm, cnt_hbm, chunk_v, acc_v, slab_sh):
    c = lax.axis_index("c")
    s = lax.axis_index("s")
    b = c * 8 + s // 2
    h = s % 2

    lane = lax.iota(jnp.int32, LANES)
    zeros = jnp.zeros((LANES,), jnp.int32)
    ones = jnp.ones((LANES,), jnp.int32)

    acc = jnp.zeros((LANES,), jnp.int32)
    for k in range(N_CH):
        row0 = h * (RSC // 2) + k * CH
        pltpu.sync_copy(sent_hbm.at[b, pl.ds(row0, CH), pl.ds(0, 128)],
                        chunk_v)

        def step(i, a):
            rows = 2 * (i * LANES + lane) + 1
            vals = plsc.load_gather(chunk_v, [rows, zeros])
            return a + jnp.where(vals != 0.0, ones, zeros)

        acc = lax.fori_loop(0, STEPS, step, acc)

    acc_v[...] = acc
    pltpu.sync_copy(acc_v, slab_sh.at[s // 2, 0, pl.ds(h * LANES, LANES)])
    plsc.subcore_barrier()

    out0 = pl.multiple_of(c * 8, 8)

    @pl.when(s == 0)
    def _():
        pltpu.sync_copy(slab_sh, cnt_hbm.at[pl.ds(out0, 8)])


def _tc_count_body(*refs):
    (*x_refs, cnt_ref) = refs
    rows = lax.broadcasted_iota(jnp.int32, (RB, 128), 0)
    odd = rows % 2 == 1
    nz = jnp.zeros((RB, 128), jnp.int32)
    for x_ref in x_refs:
        nz += jnp.where((x_ref[0] != 0.0) & odd, 1, 0)
    cnt_ref[0] = jnp.broadcast_to(jnp.sum(nz, axis=0)[None, :], (8, 128))


def _sc_gather_body(sent_hbm, sc_cnt_hbm, tc_cnt_hbm, a_hat_hbm, a_n_hbm,
                    scv, tcv, block_v, rows_sh):
    c = lax.axis_index("c")
    s = lax.axis_index("s")
    b = c * 8 + s // 2
    h = s % 2

    pltpu.sync_copy(sc_cnt_hbm.at[b], scv)
    pltpu.sync_copy(tc_cnt_hbm.at[b], tcv)
    lane = lax.iota(jnp.int32, LANES)
    count = (jnp.sum(scv[0, 0:LANES]) + jnp.sum(scv[0, LANES:2 * LANES])
             + jnp.sum(jnp.where(lane == 0, tcv[0, 0:LANES], 0)))

    r = count - 2 + h
    r = jnp.where(r < 0, r + S, r)
    rb = pl.multiple_of((r // 8) * 8, 8)
    pltpu.sync_copy(sent_hbm.at[b, pl.ds(rb, 8), :], block_v)
    pltpu.sync_copy(block_v.at[r - rb], rows_sh.at[h, s // 2])
    plsc.subcore_barrier()

    out0 = pl.multiple_of(c * 8, 8)

    @pl.when(s == 0)
    def _():
        pltpu.sync_copy(rows_sh.at[0], a_hat_hbm.at[pl.ds(out0, 8), :])

    @pl.when(s == 1)
    def _():
        pltpu.sync_copy(rows_sh.at[1], a_n_hbm.at[pl.ds(out0, 8), :])


@jax.jit
def kernel(sent_emb, entity_emb):
    del entity_emb

    sc_counts = pl.kernel(
        _sc_count_body,
        out_type=jax.ShapeDtypeStruct((B, 8, 128), jnp.int32),
        mesh=plsc.VectorSubcoreMesh(core_axis_name="c", subcore_axis_name="s"),
        scratch_types=[
            pltpu.VMEM((CH, 128), jnp.float32),
            pltpu.VMEM((LANES,), jnp.int32),
            pltpu.VMEM_SHARED((8, 8, 128), jnp.int32),
        ],
        compiler_params=pltpu.CompilerParams(needs_layout_passes=False),
    )(sent_emb)

    tc_counts = pl.pallas_call(
        _tc_count_body,
        grid=(B,),
        in_specs=[
            pl.BlockSpec((1, RB, 128), lambda b, j=j: (b, TC_BLK0 + j, 0))
            for j in range(N_STREAMS)
        ],
        out_specs=pl.BlockSpec((1, 8, 128), lambda b: (b, 0, 0)),
        out_shape=jax.ShapeDtypeStruct((B, 8, 128), jnp.int32),
    )(*([sent_emb] * N_STREAMS))

    out_row = jax.ShapeDtypeStruct((B, D), jnp.float32)
    sent_a_hat_n, sent_a_n = pl.kernel(
        _sc_gather_body,
        out_type=(out_row, out_row),
        mesh=plsc.VectorSubcoreMesh(core_axis_name="c", subcore_axis_name="s"),
        scratch_types=[
            pltpu.VMEM((8, 128), jnp.int32),
            pltpu.VMEM((8, 128), jnp.int32),
            pltpu.VMEM((8, D), jnp.float32),
            pltpu.VMEM_SHARED((2, 8, D), jnp.float32),
        ],
        compiler_params=pltpu.CompilerParams(needs_layout_passes=False),
    )(sent_emb, sc_counts, tc_counts)
    return sent_a_hat_n, sent_a_n, jnp.asarray(0.0, dtype=jnp.float32)


# speedup vs baseline: 2.3813x; 1.0002x over previous
import jax
import jax.numpy as jnp
from jax import lax
from jax.experimental import pallas as pl
from jax.experimental.pallas import tpu as pltpu
from jax.experimental.pallas import tpu_sc as plsc

B = 16
S = 4096
D = 1024
LANES = 16
RSC = 2048
CH = 512
N_CH = (RSC // 2) // CH
STEPS = (CH // 2) // LANES
RB = 128
N_STREAMS = 16
TC_BLK0 = RSC // RB


def _sc_count_body(sent_hbm, cnt_hbm, chunk_v, acc_v, slab_sh):
    c = lax.axis_index("c")
    s = lax.axis_index("s")
    b = c * 8 + s // 2
    h = s % 2

    lane = lax.iota(jnp.int32, LANES)
    zeros = jnp.zeros((LANES,), jnp.int32)
    ones = jnp.ones((LANES,), jnp.int32)

    acc = jnp.zeros((LANES,), jnp.int32)
    for k in range(N_CH):
        row0 = h * (RSC // 2) + k * CH
        pltpu.sync_copy(sent_hbm.at[b, pl.ds(row0, CH), pl.ds(0, 128)],
                        chunk_v)

        def step(i, a):
            rows = 2 * (i * LANES + lane) + 1
            vals = plsc.load_gather(chunk_v, [rows, zeros])
            return a + jnp.where(vals != 0.0, ones, zeros)

        acc = lax.fori_loop(0, STEPS, step, acc)

    acc_v[...] = acc
    pltpu.sync_copy(acc_v, slab_sh.at[s // 2, 0, pl.ds(h * LANES, LANES)])
    plsc.subcore_barrier()

    out0 = pl.multiple_of(c * 8, 8)

    @pl.when(s == 0)
    def _():
        pltpu.sync_copy(slab_sh, cnt_hbm.at[pl.ds(out0, 8)])


def _tc_count_body(*refs):
    (*x_refs, cnt_ref) = refs
    rows = lax.broadcasted_iota(jnp.int32, (RB, 128), 0)
    odd = rows % 2 == 1
    nz = jnp.zeros((RB, 128), jnp.int32)
    for x_ref in x_refs:
        nz += jnp.where((x_ref[0] != 0.0) & odd, 1, 0)
    cnt_ref[0] = jnp.broadcast_to(jnp.sum(nz, axis=0)[None, :], (8, 128))


def _sc_gather_body(sent_hbm, sc_cnt_hbm, tc_cnt_hbm, a_hat_hbm, a_n_hbm,
                    scv, tcv, block_v, rows_sh):
    c = lax.axis_index("c")
    s = lax.axis_index("s")
    b = c * 8 + s // 2
    h = s % 2

    pltpu.sync_copy(sc_cnt_hbm.at[b], scv)
    pltpu.sync_copy(tc_cnt_hbm.at[b], tcv)
    lane = lax.iota(jnp.int32, LANES)
    count = (jnp.sum(scv[0, 0:LANES]) + jnp.sum(scv[0, LANES:2 * LANES])
             + jnp.sum(jnp.where(lane == 0, tcv[0, 0:LANES], 0)))

    r = count - 2 + h
    r = jnp.where(r < 0, r + S, r)
    rb = pl.multiple_of((r // 8) * 8, 8)
    pltpu.sync_copy(sent_hbm.at[b, pl.ds(rb, 8), :], block_v)
    pltpu.sync_copy(block_v.at[r - rb], rows_sh.at[h, s // 2])
    plsc.subcore_barrier()

    out0 = pl.multiple_of(c * 8, 8)

    @pl.when(s == 0)
    def _():
        pltpu.sync_copy(rows_sh.at[0], a_hat_hbm.at[pl.ds(out0, 8), :])

    @pl.when(s == 1)
    def _():
        pltpu.sync_copy(rows_sh.at[1], a_n_hbm.at[pl.ds(out0, 8), :])


@jax.jit
def kernel(sent_emb, entity_emb):
    del entity_emb

    sc_counts = pl.kernel(
        _sc_count_body,
        out_type=jax.ShapeDtypeStruct((B, 8, 128), jnp.int32),
        mesh=plsc.VectorSubcoreMesh(core_axis_name="c", subcore_axis_name="s"),
        scratch_types=[
            pltpu.VMEM((CH, 128), jnp.float32),
            pltpu.VMEM((LANES,), jnp.int32),
            pltpu.VMEM_SHARED((8, 8, 128), jnp.int32),
        ],
        compiler_params=pltpu.CompilerParams(needs_layout_passes=False),
    )(sent_emb)

    tc_counts = pl.pallas_call(
        _tc_count_body,
        grid=(B,),
        in_specs=[
            pl.BlockSpec((1, RB, 128), lambda b, j=j: (b, TC_BLK0 + j, 0))
            for j in range(N_STREAMS)
        ],
        out_specs=pl.BlockSpec((1, 8, 128), lambda b: (b, 0, 0)),
        out_shape=jax.ShapeDtypeStruct((B, 8, 128), jnp.int32),
    )(*([sent_emb] * N_STREAMS))

    out_row = jax.ShapeDtypeStruct((B, D), jnp.float32)
    sent_a_hat_n, sent_a_n = pl.kernel(
        _sc_gather_body,
        out_type=(out_row, out_row),
        mesh=plsc.VectorSubcoreMesh(core_axis_name="c", subcore_axis_name="s"),
        scratch_types=[
            pltpu.VMEM((8, 128), jnp.int32),
            pltpu.VMEM((8, 128), jnp.int32),
            pltpu.VMEM((8, D), jnp.float32),
            pltpu.VMEM_SHARED((2, 8, D), jnp.float32),
        ],
        compiler_params=pltpu.CompilerParams(needs_layout_passes=False),
    )(sent_emb, sc_counts, tc_counts)
    return sent_a_hat_n, sent_a_n, jnp.asarray(0.0, dtype=jnp.float32)
